# baseline (speedup 1.0000x reference)
import jax
import jax.numpy as jnp
from jax import lax
from jax.experimental import pallas as pl
from jax.experimental.pallas import tpu as pltpu
from jax.experimental.pallas import tpu_sc as plsc

_B, _F, _C, _N = 128, 512, 3, 66
_G = (_C * _N) // 2
_RC = 8
_NCHUNK = _B // _RC
_UNITS = _G * _NCHUNK
_NW = 32
_NB = 8
_PF = 3
_K = 56


def _pair_slab_body(x_hbm, o_hbm, *scr):
    ev = scr[0:8]
    od = scr[8:16]
    se = scr[16:24]
    sd = scr[24:32]
    so = scr[32:40]
    wid = lax.axis_index("s") * 2 + lax.axis_index("c")

    def unit_coords(k):
        u = wid + k * _NW
        g = lax.shift_right_logical(u, 4)
        r0 = lax.bitwise_and(u, 15) * _RC
        return u, g, r0

    def in_even(k, s):
        _, g, r0 = unit_coords(k)
        return pltpu.make_async_copy(
            x_hbm.at[g, 0, pl.ds(r0, _RC), :], ev[s], se[s])

    def in_odd(k, s):
        _, g, r0 = unit_coords(k)
        return pltpu.make_async_copy(
            x_hbm.at[g, 1, pl.ds(r0, _RC), :], od[s], sd[s])

    def out_copy(k, s):
        _, g, r0 = unit_coords(k)
        return pltpu.make_async_copy(
            ev[s], o_hbm.at[g, pl.ds(r0, _RC), :], so[s])

    def accumulate(od_b, ev_b):
        def row(r, carry):
            for c in range(_F // 16):
                sl = pl.ds(c * 16, 16)
                plsc.addupdate(ev_b.at[r, sl], od_b[r, sl])
            return carry
        lax.fori_loop(0, _RC, row, 0)

    for k0 in range(_PF):
        in_even(k0, k0).start()
        in_odd(k0, k0).start()

    def step_block(p, carry):
        for b in range(_NB):
            k = p * _NB + b
            u = wid + k * _NW
            valid = u < _UNITS
            pf = u + _PF * _NW < _UNITS
            kp = k + _PF
            kd = lax.max(k - 5, 0)

            @pl.when(valid)
            def _wait_in():
                in_even(k, b).wait()
                in_odd(k, b).wait()

            @pl.when((k >= 5) & pf)
            def _drain_out():
                out_copy(kd, (b + _PF) % _NB).wait()

            @pl.when(pf)
            def _prefetch():
                in_even(kp, (b + _PF) % _NB).start()
                in_odd(kp, (b + _PF) % _NB).start()

            @pl.when(valid)
            def _go():
                accumulate(od[b], ev[b])
                out_copy(k, b).start()
        return carry

    lax.fori_loop(0, _K // _NB, step_block, 0)

    for m in range(41, 50):
        u_m = wid + m * _NW

        @pl.when((u_m + (5 + _PF) * _NW >= _UNITS) & (u_m < _UNITS))
        def _final_drain():
            out_copy(m, m % _NB).wait()


_pair_slab = pl.kernel(
    _pair_slab_body,
    out_type=jax.ShapeDtypeStruct((_G, _B, _F), jnp.float32),
    mesh=plsc.VectorSubcoreMesh(core_axis_name="c", subcore_axis_name="s"),
    compiler_params=pltpu.CompilerParams(
        needs_layout_passes=False, skip_device_barrier=True),
    scratch_types=(
        [pltpu.VMEM((_RC, _F), jnp.float32) for _ in range(16)]
        + [pltpu.SemaphoreType.DMA for _ in range(24)]
    ),
)


def kernel(x):
    xt = x.transpose(2, 3, 0, 1).reshape(_G, 2, _B, _F)
    out = _pair_slab(xt)
    return out.reshape(_C, _N // 2, _B, _F).transpose(2, 3, 0, 1)
